# Initial kernel scaffold; baseline (speedup 1.0000x reference)
import jax
import jax.numpy as jnp
from jax.experimental import pallas as pl


def kernel(x, W):
    raise NotImplementedError("write your pallas kernel here")


# speedup vs baseline: 1.6452x; 1.6452x over previous
import functools

import jax
import jax.numpy as jnp
from jax.experimental import pallas as pl
from jax.experimental.pallas import tpu as pltpu

_K = 8
_TAU = 0.7
_T = 256


def _gating_kernel(x_ref, w_ref, disp_ref, comb_ref, loss_ref, carry_ref,
                   *, cap, nblocks, G):
    i = pl.program_id(0)

    @pl.when(i == 0)
    def _init():
        carry_ref[...] = jnp.zeros_like(carry_ref)

    x = x_ref[...]
    w = w_ref[...]
    gates = jnp.dot(x, w, preferred_element_type=jnp.float32)

    m = jnp.max(gates, axis=-1, keepdims=True)
    ex = jnp.exp(gates - m)
    probs = ex / jnp.sum(ex, axis=-1, keepdims=True)

    colid = jax.lax.broadcasted_iota(jnp.int32, (_T, G), 1)

    work = probs
    csum = jnp.zeros((_T, 1), jnp.float32)
    onehots = jnp.zeros((_T, G), jnp.float32)
    vmap_raw = jnp.zeros((_T, G), jnp.float32)
    renorm = jnp.zeros((_T, 1), jnp.float32)
    for k in range(_K):
        mk = jnp.max(work, axis=-1, keepdims=True)
        ismax = work == mk
        idx = jnp.min(jnp.where(ismax, colid, G), axis=-1, keepdims=True)
        oh = (colid == idx).astype(jnp.float32)
        work = jnp.where(oh > 0.5, -1.0, work)
        csum = csum + mk
        if k == 0:
            keep = jnp.ones((_T, 1), jnp.float32)
        else:
            keep = (csum < _TAU).astype(jnp.float32)
        onehots = onehots + oh * keep
        vmap_raw = vmap_raw + oh * (mk * keep)
        renorm = renorm + mk * keep
    v_map = vmap_raw / jnp.maximum(renorm, 1e-7)

    rowid_t = jax.lax.broadcasted_iota(jnp.int32, (_T, _T), 0)
    colid_t = jax.lax.broadcasted_iota(jnp.int32, (_T, _T), 1)
    lt = (rowid_t > colid_t).astype(jnp.float32)
    excl = jnp.dot(lt, onehots, preferred_element_type=jnp.float32)
    rank = excl + carry_ref[...]

    carry_ref[...] += jnp.sum(onehots, axis=0, keepdims=True)

    placed = (onehots > 0.5) & (rank < cap - 0.5)
    r_map = jnp.where(placed, rank, -1.0)

    GC = G * cap
    rowg = jax.lax.broadcasted_iota(jnp.int32, (G, GC), 0)
    colj = jax.lax.broadcasted_iota(jnp.int32, (G, GC), 1)
    expand = (rowg == colj // cap).astype(jnp.float32)
    r_exp = jnp.dot(r_map, expand, preferred_element_type=jnp.float32)
    v_exp = jnp.dot(v_map, expand, preferred_element_type=jnp.float32)

    slot = (jax.lax.broadcasted_iota(jnp.int32, (1, GC), 1) % cap
            ).astype(jnp.float32)
    disp = (jnp.abs(r_exp - slot) < 0.5).astype(jnp.float32)
    disp_ref[...] = disp
    comb_ref[...] = v_exp * disp

    @pl.when(i == nblocks - 1)
    def _loss():
        usage = jnp.minimum(carry_ref[...], float(cap))
        mu = jnp.mean(usage)
        l = jnp.mean((usage - mu) ** 2) / (mu + 1e-8)
        loss_ref[0, 0] = jnp.where(jnp.sum(usage) > 0, l, 0.0)


def kernel(x, W):
    b, n, d = x.shape
    G = W.shape[1]
    cap = max(min(n, int(n * 1.25 / G)), 4)
    BN = b * n
    nblocks = BN // _T
    x2 = x.reshape(BN, d)
    disp, comb, loss = pl.pallas_call(
        functools.partial(_gating_kernel, cap=cap, nblocks=nblocks, G=G),
        grid=(nblocks,),
        in_specs=[
            pl.BlockSpec((_T, d), lambda i: (i, 0)),
            pl.BlockSpec((d, G), lambda i: (0, 0)),
        ],
        out_specs=[
            pl.BlockSpec((_T, G * cap), lambda i: (i, 0)),
            pl.BlockSpec((_T, G * cap), lambda i: (i, 0)),
            pl.BlockSpec(memory_space=pltpu.SMEM),
        ],
        out_shape=[
            jax.ShapeDtypeStruct((BN, G * cap), jnp.float32),
            jax.ShapeDtypeStruct((BN, G * cap), jnp.float32),
            jax.ShapeDtypeStruct((1, 1), jnp.float32),
        ],
        scratch_shapes=[pltpu.VMEM((1, G), jnp.float32)],
        compiler_params=pltpu.CompilerParams(
            dimension_semantics=("arbitrary",)),
    )(x2, W)
    return (disp.reshape(b, n, G, cap), comb.reshape(b, n, G, cap),
            loss.reshape(()))


# speedup vs baseline: 2.0981x; 1.2753x over previous
import functools

import jax
import jax.numpy as jnp
from jax.experimental import pallas as pl
from jax.experimental.pallas import tpu as pltpu

_K = 8
_TAU = 0.7
_T = 256


def _gating_kernel(x_ref, w_ref, disp_ref, comb_ref, loss_ref, carry_ref,
                   *, cap, nblocks, G):
    i = pl.program_id(0)

    @pl.when(i == 0)
    def _init():
        carry_ref[...] = jnp.zeros_like(carry_ref)

    x = x_ref[...]
    w = w_ref[...]
    gates = jnp.dot(x, w, preferred_element_type=jnp.float32)

    m = jnp.max(gates, axis=-1, keepdims=True)
    ex = jnp.exp(gates - m)
    probs = ex / jnp.sum(ex, axis=-1, keepdims=True)

    colid = jax.lax.broadcasted_iota(jnp.int32, (_T, G), 1)

    work = probs
    csum = jnp.zeros((_T, 1), jnp.float32)
    onehots = jnp.zeros((_T, G), jnp.float32)
    vmap_raw = jnp.zeros((_T, G), jnp.float32)
    renorm = jnp.zeros((_T, 1), jnp.float32)
    for k in range(_K):
        mk = jnp.max(work, axis=-1, keepdims=True)
        ismax = work == mk
        idx = jnp.min(jnp.where(ismax, colid, G), axis=-1, keepdims=True)
        oh = (colid == idx).astype(jnp.float32)
        work = jnp.where(oh > 0.5, -1.0, work)
        csum = csum + mk
        if k == 0:
            keep = jnp.ones((_T, 1), jnp.float32)
        else:
            keep = (csum < _TAU).astype(jnp.float32)
        onehots = onehots + oh * keep
        vmap_raw = vmap_raw + oh * (mk * keep)
        renorm = renorm + mk * keep
    v_map = vmap_raw / jnp.maximum(renorm, 1e-7)

    rowid_t = jax.lax.broadcasted_iota(jnp.int32, (_T, _T), 0)
    colid_t = jax.lax.broadcasted_iota(jnp.int32, (_T, _T), 1)
    lt = (rowid_t > colid_t).astype(jnp.float32)
    excl = jnp.dot(lt, onehots, preferred_element_type=jnp.float32)
    rank = excl + carry_ref[...]

    carry_ref[...] += jnp.sum(onehots, axis=0, keepdims=True)

    placed = (onehots > 0.5) & (rank < cap - 0.5)
    r_map = jnp.where(placed, rank, -1.0)

    ci = jax.lax.broadcasted_iota(jnp.int32, (_T, G, cap), 2)
    r_i = r_map.astype(jnp.int32)
    disp = (ci == r_i[:, :, None]).astype(jnp.float32)
    disp_ref[...] = disp
    comb_ref[...] = v_map[:, :, None] * disp

    @pl.when(i == nblocks - 1)
    def _loss():
        usage = jnp.minimum(carry_ref[...], float(cap))
        mu = jnp.mean(usage)
        l = jnp.mean((usage - mu) ** 2) / (mu + 1e-8)
        loss_ref[0, 0] = jnp.where(jnp.sum(usage) > 0, l, 0.0)


def kernel(x, W):
    b, n, d = x.shape
    G = W.shape[1]
    cap = max(min(n, int(n * 1.25 / G)), 4)
    BN = b * n
    nblocks = BN // _T
    x2 = x.reshape(BN, d)
    disp, comb, loss = pl.pallas_call(
        functools.partial(_gating_kernel, cap=cap, nblocks=nblocks, G=G),
        grid=(nblocks,),
        in_specs=[
            pl.BlockSpec((_T, d), lambda i: (i, 0)),
            pl.BlockSpec((d, G), lambda i: (0, 0)),
        ],
        out_specs=[
            pl.BlockSpec((_T, G, cap), lambda i: (i, 0, 0)),
            pl.BlockSpec((_T, G, cap), lambda i: (i, 0, 0)),
            pl.BlockSpec(memory_space=pltpu.SMEM),
        ],
        out_shape=[
            jax.ShapeDtypeStruct((BN, G, cap), jnp.float32),
            jax.ShapeDtypeStruct((BN, G, cap), jnp.float32),
            jax.ShapeDtypeStruct((1, 1), jnp.float32),
        ],
        scratch_shapes=[pltpu.VMEM((1, G), jnp.float32)],
        compiler_params=pltpu.CompilerParams(
            dimension_semantics=("arbitrary",)),
    )(x2, W)
    return (disp.reshape(b, n, G, cap), comb.reshape(b, n, G, cap),
            loss.reshape(()))
